# baseline (device time: 35605 ns/iter reference)
import jax
import jax.numpy as jnp
from jax import lax
from jax.experimental import pallas as pl
from jax.experimental.pallas import tpu as pltpu

N_DEV = 4
DH = 64
B = 2
SQ = 256
D = 768
HALF = D // 2

CHUNKS = [(0, 0, 256, 0), (1, 0, 192, 256), (1, 192, 64, 448)]


def _fused(xb, Wq, Wo, K, V):
    Hq = Wq.shape[1] // DH

    def body(x_ref, wq_ref, wo_ref, k_any, v_any, out_ref,
             s1L, s1R, s2L, s2R, r1L, r1R, r2L, r2R, o_buf,
             kslab, vslab, k_ref, v_ref,
             sph1, rph1, sph2, rph2, dma_sems):
        my = lax.axis_index("i")
        p1 = my ^ 1
        p2 = 3 - my
        H0 = my * Hq * DH

        slab_dmas = []
        for i, (any_ref, slab) in enumerate(
                ((k_any, kslab), (v_any, vslab))):
            for b in range(B):
                d = pltpu.make_async_copy(
                    any_ref.at[b, :, pl.ds(H0, Hq * DH)],
                    slab.at[b], dma_sems.at[2 * i + b],
                )
                d.start()
                slab_dmas.append(d)

        barrier_sem = pltpu.get_barrier_semaphore()
        for nbr in (p1, p2):
            pl.semaphore_signal(
                barrier_sem, inc=1,
                device_id=(nbr,), device_id_type=pl.DeviceIdType.MESH,
            )
        pl.semaphore_wait(barrier_sem, 2)

        kv_ready = [False, False]

        def ensure_kv(b):
            if not kv_ready[b]:
                slab_dmas[b].wait()
                slab_dmas[B + b].wait()
                k_ref[b] = kslab[b].astype(jnp.bfloat16)
                v_ref[b] = vslab[b].astype(jnp.bfloat16)
                kv_ready[b] = True

        def rdma(src, dst, ss, rs, tgt):
            return pltpu.make_async_remote_copy(
                src_ref=src, dst_ref=dst, send_sem=ss, recv_sem=rs,
                device_id=(tgt,), device_id_type=pl.DeviceIdType.MESH,
            )

        def compute_chunk(c):
            b, q0, nr, _ = CHUNKS[c]
            q = (jnp.dot(
                x_ref[b, pl.ds(q0, nr), :], wq_ref[...],
                preferred_element_type=jnp.float32,
            ) * 0.125).astype(jnp.bfloat16)
            ensure_kv(b)
            for h in range(Hq):
                qh = q[:, DH * h:DH * (h + 1)]
                s = lax.dot_general(
                    qh, k_ref[b, :, pl.ds(DH * h, DH)],
                    (((1,), (1,)), ((), ())),
                    preferred_element_type=jnp.float32,
                )
                p32 = jnp.exp(s)
                rl = 1.0 / jnp.sum(p32, axis=1, keepdims=True)
                ov = jnp.dot(p32.astype(jnp.bfloat16),
                             v_ref[b, :, pl.ds(DH * h, DH)],
                             preferred_element_type=jnp.float32)
                o_buf[pl.ds(0, nr), pl.ds(DH * h, DH)] = (
                    ov * rl
                ).astype(jnp.bfloat16)
            return jnp.dot(o_buf[pl.ds(0, nr), :], wo_ref[...],
                           preferred_element_type=jnp.float32)

        def start_ph1(c):
            b, q0, nr, r0 = CHUNKS[c]
            part = compute_chunk(c)
            s1L[pl.ds(r0, nr), :] = part[:, 0:HALF].astype(jnp.bfloat16)
            s1R[pl.ds(r0, nr), :] = part[:, HALF:D].astype(jnp.bfloat16)
            dL = rdma(s1L.at[pl.ds(r0, nr), :],
                      r1L.at[pl.ds(r0, nr), :],
                      sph1.at[2 * c], rph1.at[2 * c], p1)
            dR = rdma(s1R.at[pl.ds(r0, nr), :],
                      r1R.at[pl.ds(r0, nr), :],
                      sph1.at[2 * c + 1], rph1.at[2 * c + 1], p2)
            dL.start()
            dR.start()
            return part, dL, dR

        def start_ph2(c, st):
            part, dL, dR = st
            _, _, nr, r0 = CHUNKS[c]
            dL.wait()
            dR.wait()
            pairL = part[:, 0:HALF] + r1L[pl.ds(r0, nr), :].astype(jnp.float32)
            pairR = part[:, HALF:D] + r1R[pl.ds(r0, nr), :].astype(jnp.float32)
            s2L[pl.ds(r0, nr), :] = pairL.astype(jnp.bfloat16)
            s2R[pl.ds(r0, nr), :] = pairR.astype(jnp.bfloat16)
            eL = rdma(s2L.at[pl.ds(r0, nr), :],
                      r2L.at[pl.ds(r0, nr), :],
                      sph2.at[2 * c], rph2.at[2 * c], p2)
            eR = rdma(s2R.at[pl.ds(r0, nr), :],
                      r2R.at[pl.ds(r0, nr), :],
                      sph2.at[2 * c + 1], rph2.at[2 * c + 1], p1)
            eL.start()
            eR.start()
            return pairL, pairR, eL, eR

        def finish(c, st):
            pairL, pairR, eL, eR = st
            _, _, nr, r0 = CHUNKS[c]
            eL.wait()
            eR.wait()
            out_ref[pl.ds(r0, nr), pl.ds(0, HALF)] = (
                pairL + r2L[pl.ds(r0, nr), :].astype(jnp.float32)
            ).astype(jnp.bfloat16)
            out_ref[pl.ds(r0, nr), pl.ds(HALF, HALF)] = (
                pairR + r2R[pl.ds(r0, nr), :].astype(jnp.float32)
            ).astype(jnp.bfloat16)

        st0 = start_ph1(0)
        st1 = start_ph1(1)
        f0 = start_ph2(0, st0)
        st2 = start_ph1(2)
        f1 = start_ph2(1, st1)
        finish(0, f0)
        f2 = start_ph2(2, st2)
        finish(1, f1)
        finish(2, f2)

    return pl.pallas_call(
        body,
        out_shape=jax.ShapeDtypeStruct((B * SQ, D), jnp.bfloat16),
        in_specs=[pl.BlockSpec(memory_space=pltpu.VMEM)] * 3
        + [pl.BlockSpec(memory_space=pl.ANY)] * 2,
        out_specs=pl.BlockSpec(memory_space=pltpu.VMEM),
        scratch_shapes=[
            pltpu.VMEM((B * SQ, HALF), jnp.bfloat16),
            pltpu.VMEM((B * SQ, HALF), jnp.bfloat16),
            pltpu.VMEM((B * SQ, HALF), jnp.bfloat16),
            pltpu.VMEM((B * SQ, HALF), jnp.bfloat16),
            pltpu.VMEM((B * SQ, HALF), jnp.bfloat16),
            pltpu.VMEM((B * SQ, HALF), jnp.bfloat16),
            pltpu.VMEM((B * SQ, HALF), jnp.bfloat16),
            pltpu.VMEM((B * SQ, HALF), jnp.bfloat16),
            pltpu.VMEM((SQ, 8 * DH), jnp.bfloat16),
            pltpu.VMEM((B, 512, 8 * DH), jnp.float32),
            pltpu.VMEM((B, 512, 8 * DH), jnp.float32),
            pltpu.VMEM((B, 512, 8 * DH), jnp.bfloat16),
            pltpu.VMEM((B, 512, 8 * DH), jnp.bfloat16),
            pltpu.SemaphoreType.DMA((8,)),
            pltpu.SemaphoreType.DMA((8,)),
            pltpu.SemaphoreType.DMA((8,)),
            pltpu.SemaphoreType.DMA((8,)),
            pltpu.SemaphoreType.DMA((4,)),
        ],
        compiler_params=pltpu.CompilerParams(collective_id=0),
    )(xb, Wq, Wo, K, V)


def kernel(x, Wq, Wo, K_ext, V_ext):
    my = lax.axis_index("i")
    Hq = Wq.shape[1] // DH

    xb = x.astype(jnp.bfloat16)
    Skv = K_ext.shape[1]
    Hkv = K_ext.shape[2]
    K = K_ext.reshape(B, Skv, Hkv * DH)
    V = V_ext.reshape(B, Skv, Hkv * DH)

    out = _fused(
        xb, Wq.astype(jnp.bfloat16), Wo.astype(jnp.bfloat16), K, V
    )
    return out.reshape(B, SQ, D)


# device time: 21678 ns/iter; 1.6424x vs baseline; 1.6424x over previous
import jax
import jax.numpy as jnp
from jax import lax
from jax.experimental import pallas as pl
from jax.experimental.pallas import tpu as pltpu

N_DEV = 4
DH = 64
B = 2
SQ = 256
D = 768
HALF = D // 2

CHUNKS = [(0, 0, 256, 0), (1, 0, 192, 256), (1, 192, 64, 448)]


def _fused(xb, Wq, Wo, K, V):
    Hq = K.shape[2] // DH

    def body(x_ref, wq_ref, wo_ref, k_ref, v_ref, out_ref,
             s1L, s1R, s2L, s2R, r1L, r1R, r2L, r2R, o_buf,
             sph1, rph1, sph2, rph2):
        my = lax.axis_index("i")
        p1 = my ^ 1
        p2 = 3 - my

        def entry_barrier():
            barrier_sem = pltpu.get_barrier_semaphore()
            for nbr in (p1, p2):
                pl.semaphore_signal(
                    barrier_sem, inc=1,
                    device_id=(nbr,), device_id_type=pl.DeviceIdType.MESH,
                )
            pl.semaphore_wait(barrier_sem, 2)

        def rdma(src, dst, ss, rs, tgt):
            return pltpu.make_async_remote_copy(
                src_ref=src, dst_ref=dst, send_sem=ss, recv_sem=rs,
                device_id=(tgt,), device_id_type=pl.DeviceIdType.MESH,
            )

        def compute_chunk(c):
            b, q0, nr, _ = CHUNKS[c]
            q = (jnp.dot(
                x_ref[b, pl.ds(q0, nr), :], wq_ref[...],
                preferred_element_type=jnp.float32,
            ) * 0.125).astype(jnp.bfloat16)
            for h in range(Hq):
                qh = q[:, DH * h:DH * (h + 1)]
                s = lax.dot_general(
                    qh, k_ref[b, :, pl.ds(DH * h, DH)],
                    (((1,), (1,)), ((), ())),
                    preferred_element_type=jnp.float32,
                )
                p32 = jnp.exp(s)
                rl = 1.0 / jnp.sum(p32, axis=1, keepdims=True)
                ov = jnp.dot(p32.astype(jnp.bfloat16),
                             v_ref[b, :, pl.ds(DH * h, DH)],
                             preferred_element_type=jnp.float32)
                o_buf[pl.ds(0, nr), pl.ds(DH * h, DH)] = (
                    ov * rl
                ).astype(jnp.bfloat16)
            return jnp.dot(o_buf[pl.ds(0, nr), :], wo_ref[...],
                           preferred_element_type=jnp.float32)

        def start_ph1(c):
            b, q0, nr, r0 = CHUNKS[c]
            part = compute_chunk(c)
            s1L[pl.ds(r0, nr), :] = part[:, 0:HALF].astype(jnp.bfloat16)
            s1R[pl.ds(r0, nr), :] = part[:, HALF:D].astype(jnp.bfloat16)
            if c == 0:
                entry_barrier()
            dL = rdma(s1L.at[pl.ds(r0, nr), :],
                      r1L.at[pl.ds(r0, nr), :],
                      sph1.at[2 * c], rph1.at[2 * c], p1)
            dR = rdma(s1R.at[pl.ds(r0, nr), :],
                      r1R.at[pl.ds(r0, nr), :],
                      sph1.at[2 * c + 1], rph1.at[2 * c + 1], p2)
            dL.start()
            dR.start()
            return part, dL, dR

        def start_ph2(c, st):
            part, dL, dR = st
            _, _, nr, r0 = CHUNKS[c]
            dL.wait()
            dR.wait()
            pairL = part[:, 0:HALF] + r1L[pl.ds(r0, nr), :].astype(jnp.float32)
            pairR = part[:, HALF:D] + r1R[pl.ds(r0, nr), :].astype(jnp.float32)
            s2L[pl.ds(r0, nr), :] = pairL.astype(jnp.bfloat16)
            s2R[pl.ds(r0, nr), :] = pairR.astype(jnp.bfloat16)
            eL = rdma(s2L.at[pl.ds(r0, nr), :],
                      r2L.at[pl.ds(r0, nr), :],
                      sph2.at[2 * c], rph2.at[2 * c], p2)
            eR = rdma(s2R.at[pl.ds(r0, nr), :],
                      r2R.at[pl.ds(r0, nr), :],
                      sph2.at[2 * c + 1], rph2.at[2 * c + 1], p1)
            eL.start()
            eR.start()
            return pairL, pairR, eL, eR

        def finish(c, st):
            pairL, pairR, eL, eR = st
            _, _, nr, r0 = CHUNKS[c]
            eL.wait()
            eR.wait()
            out_ref[pl.ds(r0, nr), pl.ds(0, HALF)] = (
                pairL + r2L[pl.ds(r0, nr), :].astype(jnp.float32)
            ).astype(jnp.bfloat16)
            out_ref[pl.ds(r0, nr), pl.ds(HALF, HALF)] = (
                pairR + r2R[pl.ds(r0, nr), :].astype(jnp.float32)
            ).astype(jnp.bfloat16)

        st0 = start_ph1(0)
        st1 = start_ph1(1)
        f0 = start_ph2(0, st0)
        st2 = start_ph1(2)
        f1 = start_ph2(1, st1)
        finish(0, f0)
        f2 = start_ph2(2, st2)
        finish(1, f1)
        finish(2, f2)

    return pl.pallas_call(
        body,
        out_shape=jax.ShapeDtypeStruct((B * SQ, D), jnp.bfloat16),
        in_specs=[pl.BlockSpec(memory_space=pltpu.VMEM)] * 5,
        out_specs=pl.BlockSpec(memory_space=pltpu.VMEM),
        scratch_shapes=[
            pltpu.VMEM((B * SQ, HALF), jnp.bfloat16),
            pltpu.VMEM((B * SQ, HALF), jnp.bfloat16),
            pltpu.VMEM((B * SQ, HALF), jnp.bfloat16),
            pltpu.VMEM((B * SQ, HALF), jnp.bfloat16),
            pltpu.VMEM((B * SQ, HALF), jnp.bfloat16),
            pltpu.VMEM((B * SQ, HALF), jnp.bfloat16),
            pltpu.VMEM((B * SQ, HALF), jnp.bfloat16),
            pltpu.VMEM((B * SQ, HALF), jnp.bfloat16),
            pltpu.VMEM((SQ, 8 * DH), jnp.bfloat16),
            pltpu.SemaphoreType.DMA((8,)),
            pltpu.SemaphoreType.DMA((8,)),
            pltpu.SemaphoreType.DMA((8,)),
            pltpu.SemaphoreType.DMA((8,)),
        ],
        compiler_params=pltpu.CompilerParams(collective_id=0),
    )(xb, Wq, Wo, K, V)


def kernel(x, Wq, Wo, K_ext, V_ext):
    my = lax.axis_index("i")
    Hq = Wq.shape[1] // DH

    xb = x.astype(jnp.bfloat16)
    Skv = K_ext.shape[1]
    K = lax.dynamic_slice_in_dim(K_ext, my * Hq, Hq, axis=2)
    V = lax.dynamic_slice_in_dim(V_ext, my * Hq, Hq, axis=2)
    K = K.astype(jnp.bfloat16).reshape(B, Skv, Hq * DH)
    V = V.astype(jnp.bfloat16).reshape(B, Skv, Hq * DH)

    out = _fused(
        xb, Wq.astype(jnp.bfloat16), Wo.astype(jnp.bfloat16), K, V
    )
    return out.reshape(B, SQ, D)


# device time: 21631 ns/iter; 1.6460x vs baseline; 1.0022x over previous
import jax
import jax.numpy as jnp
from jax import lax
from jax.experimental import pallas as pl
from jax.experimental.pallas import tpu as pltpu

N_DEV = 4
DH = 64
B = 2
SQ = 256
D = 768
HALF = D // 2

CHUNKS = [(0, 0, 256, 0), (1, 0, 192, 256), (1, 192, 64, 448)]


def _fused(xb, Wq, Wo, K, V):
    Hq = K.shape[2] // DH

    def body(x_ref, wq_ref, wo_ref, k_ref, v_ref, out_ref,
             s1L, s1R, s2L, s2R, r1L, r1R, r2L, r2R, o_buf,
             sph1, rph1, sph2, rph2):
        my = lax.axis_index("i")
        p1 = my ^ 1
        p2 = 3 - my

        def entry_barrier():
            barrier_sem = pltpu.get_barrier_semaphore()
            for nbr in (p1, p2):
                pl.semaphore_signal(
                    barrier_sem, inc=1,
                    device_id=(nbr,), device_id_type=pl.DeviceIdType.MESH,
                )
            pl.semaphore_wait(barrier_sem, 2)

        def rdma(src, dst, ss, rs, tgt):
            return pltpu.make_async_remote_copy(
                src_ref=src, dst_ref=dst, send_sem=ss, recv_sem=rs,
                device_id=(tgt,), device_id_type=pl.DeviceIdType.MESH,
            )

        def compute_chunk(c):
            b, q0, nr, _ = CHUNKS[c]
            q = (jnp.dot(
                x_ref[b, pl.ds(q0, nr), :], wq_ref[...],
                preferred_element_type=jnp.float32,
            ) * 0.125).astype(jnp.bfloat16)
            for h in range(Hq):
                qh = q[:, DH * h:DH * (h + 1)]
                s = lax.dot_general(
                    qh, k_ref[b, :, pl.ds(DH * h, DH)],
                    (((1,), (1,)), ((), ())),
                    preferred_element_type=jnp.float32,
                )
                p32 = jnp.exp(s)
                rl = 1.0 / jnp.sum(p32, axis=1, keepdims=True)
                ov = jnp.dot(p32.astype(jnp.bfloat16),
                             v_ref[b, :, pl.ds(DH * h, DH)],
                             preferred_element_type=jnp.float32)
                o_buf[pl.ds(0, nr), pl.ds(DH * h, DH)] = (
                    ov * rl
                ).astype(jnp.bfloat16)
            return jnp.dot(o_buf[pl.ds(0, nr), :], wo_ref[...],
                           preferred_element_type=jnp.float32)

        def start_ph1(c):
            b, q0, nr, r0 = CHUNKS[c]
            part = compute_chunk(c)
            s1L[pl.ds(r0, nr), :] = part[:, 0:HALF].astype(jnp.bfloat16)
            s1R[pl.ds(r0, nr), :] = part[:, HALF:D].astype(jnp.bfloat16)
            if c == 0:
                entry_barrier()
            dL = rdma(s1L.at[pl.ds(r0, nr), :],
                      r1L.at[pl.ds(r0, nr), :],
                      sph1.at[2 * c], rph1.at[2 * c], p1)
            dR = rdma(s1R.at[pl.ds(r0, nr), :],
                      r1R.at[pl.ds(r0, nr), :],
                      sph1.at[2 * c + 1], rph1.at[2 * c + 1], p2)
            dL.start()
            dR.start()
            return part, dL, dR

        def start_ph2(c, st):
            part, dL, dR = st
            _, _, nr, r0 = CHUNKS[c]
            dL.wait()
            pairL = part[:, 0:HALF] + r1L[pl.ds(r0, nr), :].astype(jnp.float32)
            s2L[pl.ds(r0, nr), :] = pairL.astype(jnp.bfloat16)
            eL = rdma(s2L.at[pl.ds(r0, nr), :],
                      r2L.at[pl.ds(r0, nr), :],
                      sph2.at[2 * c], rph2.at[2 * c], p2)
            eL.start()
            dR.wait()
            pairR = part[:, HALF:D] + r1R[pl.ds(r0, nr), :].astype(jnp.float32)
            s2R[pl.ds(r0, nr), :] = pairR.astype(jnp.bfloat16)
            eR = rdma(s2R.at[pl.ds(r0, nr), :],
                      r2R.at[pl.ds(r0, nr), :],
                      sph2.at[2 * c + 1], rph2.at[2 * c + 1], p1)
            eR.start()
            return pairL, pairR, eL, eR

        def finish(c, st):
            pairL, pairR, eL, eR = st
            _, _, nr, r0 = CHUNKS[c]
            eL.wait()
            out_ref[pl.ds(r0, nr), pl.ds(0, HALF)] = (
                pairL + r2L[pl.ds(r0, nr), :].astype(jnp.float32)
            ).astype(jnp.bfloat16)
            eR.wait()
            out_ref[pl.ds(r0, nr), pl.ds(HALF, HALF)] = (
                pairR + r2R[pl.ds(r0, nr), :].astype(jnp.float32)
            ).astype(jnp.bfloat16)

        st0 = start_ph1(0)
        st1 = start_ph1(1)
        f0 = start_ph2(0, st0)
        st2 = start_ph1(2)
        f1 = start_ph2(1, st1)
        finish(0, f0)
        f2 = start_ph2(2, st2)
        finish(1, f1)
        finish(2, f2)

    return pl.pallas_call(
        body,
        out_shape=jax.ShapeDtypeStruct((B * SQ, D), jnp.bfloat16),
        in_specs=[pl.BlockSpec(memory_space=pltpu.VMEM)] * 5,
        out_specs=pl.BlockSpec(memory_space=pltpu.VMEM),
        scratch_shapes=[
            pltpu.VMEM((B * SQ, HALF), jnp.bfloat16),
            pltpu.VMEM((B * SQ, HALF), jnp.bfloat16),
            pltpu.VMEM((B * SQ, HALF), jnp.bfloat16),
            pltpu.VMEM((B * SQ, HALF), jnp.bfloat16),
            pltpu.VMEM((B * SQ, HALF), jnp.bfloat16),
            pltpu.VMEM((B * SQ, HALF), jnp.bfloat16),
            pltpu.VMEM((B * SQ, HALF), jnp.bfloat16),
            pltpu.VMEM((B * SQ, HALF), jnp.bfloat16),
            pltpu.VMEM((SQ, 8 * DH), jnp.bfloat16),
            pltpu.SemaphoreType.DMA((8,)),
            pltpu.SemaphoreType.DMA((8,)),
            pltpu.SemaphoreType.DMA((8,)),
            pltpu.SemaphoreType.DMA((8,)),
        ],
        compiler_params=pltpu.CompilerParams(collective_id=0),
    )(xb, Wq, Wo, K, V)


def kernel(x, Wq, Wo, K_ext, V_ext):
    my = lax.axis_index("i")
    Hq = Wq.shape[1] // DH

    xb = x.astype(jnp.bfloat16)
    Skv = K_ext.shape[1]
    K = lax.dynamic_slice_in_dim(K_ext, my * Hq, Hq, axis=2)
    V = lax.dynamic_slice_in_dim(V_ext, my * Hq, Hq, axis=2)
    K = K.astype(jnp.bfloat16).reshape(B, Skv, Hq * DH)
    V = V.astype(jnp.bfloat16).reshape(B, Skv, Hq * DH)

    out = _fused(
        xb, Wq.astype(jnp.bfloat16), Wo.astype(jnp.bfloat16), K, V
    )
    return out.reshape(B, SQ, D)
